# baseline (speedup 1.0000x reference)
import jax
import jax.numpy as jnp
from jax.experimental import pallas as pl
from jax.experimental.pallas import tpu as pltpu

_G = 50
_GP = _G + 2
_P = _GP * _GP
_PL = 2816
_EXT = _P + 106
_OFFS = (0, 1, 2, _GP, _GP + 1, _GP + 2, 2 * _GP, 2 * _GP + 1, 2 * _GP + 2)
_CC = 32
_RD = 128
_ZD = 64
_HD = 128


def _enc_body(yx_ref, me_ref, W1_ref, b1_ref, W2_ref, b2_ref,
              We1_ref, be1_ref, We2_ref, be2_ref, out_ref):
    n = pl.program_id(1)
    row = yx_ref[0, 0]
    yext = row[:, :_EXT]
    x11 = row[:, _EXT:_EXT + 1]
    me = me_ref[...]

    bf16 = jnp.bfloat16
    ys = [yext[:, s:s + _P] for s in _OFFS]
    ms = [me[:, s:s + _P] * x11 for s in _OFFS]
    C1 = jnp.concatenate(ys + ms, axis=0).astype(bf16)
    H1 = jnp.dot(W1_ref[...], C1, preferred_element_type=jnp.float32)
    H1 = jnp.maximum(H1 + b1_ref[...], 0.0)
    maskt = me[:, 53:53 + _P]
    H1 = H1 * maskt

    z53 = jnp.zeros((_CC, 53), jnp.float32)
    H1e = jnp.concatenate([z53, H1, z53], axis=1)
    C2 = jnp.concatenate(
        [H1e[:, s:s + _P] for s in _OFFS], axis=0).astype(bf16)
    H2 = jnp.dot(W2_ref[...], C2, preferred_element_type=jnp.float32)
    H2 = jnp.maximum(H2 + b2_ref[...], 0.0)

    pooled = jax.lax.dot_general(
        maskt.astype(bf16), H2.astype(bf16), (((1,), (1,)), ((), ())),
        preferred_element_type=jnp.float32) * (1.0 / 2500.0)

    feat = jnp.concatenate([pooled, x11], axis=1).astype(bf16)
    h = jnp.maximum(jnp.dot(feat, We1_ref[...],
                            preferred_element_type=jnp.float32)
                    + be1_ref[...], 0.0)
    r_i = jnp.dot(h.astype(bf16), We2_ref[...],
                  preferred_element_type=jnp.float32) + be2_ref[...]

    @pl.when(n == 0)
    def _():
        out_ref[0] = r_i

    @pl.when(n != 0)
    def _():
        out_ref[0] = out_ref[0] + r_i


def _dec_body(rs_ref, eps_ref, xt_ref,
              Wh_ref, bh_ref, Wmu_ref, bmu_ref, Wsig_ref, bsig_ref,
              W1x_ref, W1z_ref, bd1_ref, Wd2_ref, bd2_ref,
              Wdmu_ref, bdmu_ref, Wdsig_ref, bdsig_ref,
              ymu_ref, ysig_ref, mu_ref, sig_ref):
    bf16 = jnp.bfloat16
    r = rs_ref[0] * (1.0 / 64.0)
    hr = jnp.maximum(jnp.dot(r.astype(bf16), Wh_ref[...],
                             preferred_element_type=jnp.float32)
                     + bh_ref[...], 0.0).astype(bf16)
    mu = jnp.dot(hr, Wmu_ref[...],
                 preferred_element_type=jnp.float32) + bmu_ref[...]
    sg = 0.1 + 0.9 * jax.nn.sigmoid(
        jnp.dot(hr, Wsig_ref[...],
                preferred_element_type=jnp.float32) + bsig_ref[...])
    z = mu + sg * eps_ref[0]

    xt = xt_ref[0]
    h1 = jnp.dot(xt.astype(bf16), W1x_ref[...],
                 preferred_element_type=jnp.float32)
    h1 = h1 + jnp.dot(z.astype(bf16), W1z_ref[...],
                      preferred_element_type=jnp.float32) + bd1_ref[...]
    h1 = jnp.maximum(h1, 0.0).astype(bf16)
    h2 = jnp.maximum(jnp.dot(h1, Wd2_ref[...],
                             preferred_element_type=jnp.float32)
                     + bd2_ref[...], 0.0).astype(bf16)
    ymu_ref[0] = jnp.dot(h2, Wdmu_ref[...],
                         preferred_element_type=jnp.float32) + bdmu_ref[...]
    t = jnp.dot(h2, Wdsig_ref[...],
                preferred_element_type=jnp.float32) + bdsig_ref[...]
    sp = jnp.maximum(t, 0.0) + jnp.log1p(jnp.exp(-jnp.abs(t)))
    ysig_ref[0] = 0.1 + 0.9 * sp
    mu_ref[0] = mu
    sig_ref[0] = sg


def kernel(x_context, y_context, x_target, eps,
           w1, b1, w2, b2, We1, be1, We2, be2,
           Wh, bh, Wmu, bmu, Wsig, bsig,
           Wd1, bd1, Wd2, bd2, Wdmu, bdmu, Wdsig, bdsig):
    f32 = jnp.float32
    B, NC, _ = x_context.shape
    NT = x_target.shape[1]
    YD = y_context.shape[2]

    yg = y_context.reshape(B, NC, _G, _G)
    ygp = jnp.pad(yg, ((0, 0), (0, 0), (1, 1), (1, 1)))
    yflat = ygp.reshape(B, NC, _P)
    yext = jnp.pad(yflat, ((0, 0), (0, 0), (53, 53)))
    yx = jnp.concatenate([yext, x_context], axis=2)
    yx = jnp.pad(yx, ((0, 0), (0, 0), (0, _PL - _EXT - 1)))
    yx = yx.reshape(B, NC, 1, _PL)

    m2 = jnp.pad(jnp.ones((_G, _G), f32), ((1, 1), (1, 1))).reshape(1, _P)
    maskext = jnp.pad(m2, ((0, 0), (53, _PL - _P - 53)))

    bf16 = jnp.bfloat16
    w1y = w1[:, 0].reshape(_CC, 9)
    w1x = w1[:, 1].reshape(_CC, 9)
    W1cat = jnp.concatenate([w1y, w1x], axis=1).astype(bf16)
    b1c = b1.reshape(_CC, 1)
    W2T = jnp.transpose(w2, (0, 2, 3, 1)).reshape(_CC, 9 * _CC).astype(bf16)
    b2c = b2.reshape(_CC, 1)
    be1r = be1.reshape(1, _HD)
    be2r = be2.reshape(1, _RD)

    cparams = pltpu.CompilerParams(
        dimension_semantics=("parallel", "arbitrary"),
        vmem_limit_bytes=100 * 1024 * 1024,
    )
    full = lambda shp: pl.BlockSpec(shp, lambda b, n: (0,) * len(shp))
    r_sum = pl.pallas_call(
        _enc_body,
        grid=(B, NC),
        in_specs=[
            pl.BlockSpec((1, 1, 1, _PL), lambda b, n: (b, n, 0, 0)),
            full((1, _PL)),
            full((_CC, 18)), full((_CC, 1)),
            full((_CC, 9 * _CC)), full((_CC, 1)),
            full((_CC + 1, _HD)), full((1, _HD)),
            full((_HD, _RD)), full((1, _RD)),
        ],
        out_specs=pl.BlockSpec((1, 1, _RD), lambda b, n: (b, 0, 0)),
        out_shape=jax.ShapeDtypeStruct((B, 1, _RD), f32),
        compiler_params=cparams,
    )(yx, maskext, W1cat, b1c, W2T, b2c,
      We1.astype(bf16), be1r, We2.astype(bf16), be2r)

    cparams2 = pltpu.CompilerParams(
        dimension_semantics=("parallel",),
        vmem_limit_bytes=100 * 1024 * 1024,
    )
    full1 = lambda shp: pl.BlockSpec(shp, lambda b: (0,) * len(shp))
    eps3 = eps.reshape(B, 1, _ZD)
    out_shapes = (
        jax.ShapeDtypeStruct((B, NT, YD), f32),
        jax.ShapeDtypeStruct((B, NT, YD), f32),
        jax.ShapeDtypeStruct((B, 1, _ZD), f32),
        jax.ShapeDtypeStruct((B, 1, _ZD), f32),
    )
    y_mu, y_sigma, mu_c, sigma_c = pl.pallas_call(
        _dec_body,
        grid=(B,),
        in_specs=[
            pl.BlockSpec((1, 1, _RD), lambda b: (b, 0, 0)),
            pl.BlockSpec((1, 1, _ZD), lambda b: (b, 0, 0)),
            pl.BlockSpec((1, NT, 1), lambda b: (b, 0, 0)),
            full1((_RD, _HD)), full1((1, _HD)),
            full1((_HD, _ZD)), full1((1, _ZD)),
            full1((_HD, _ZD)), full1((1, _ZD)),
            full1((1, _HD)), full1((_ZD, _HD)), full1((1, _HD)),
            full1((_HD, _HD)), full1((1, _HD)),
            full1((_HD, YD)), full1((1, YD)),
            full1((_HD, YD)), full1((1, YD)),
        ],
        out_specs=(
            pl.BlockSpec((1, NT, YD), lambda b: (b, 0, 0)),
            pl.BlockSpec((1, NT, YD), lambda b: (b, 0, 0)),
            pl.BlockSpec((1, 1, _ZD), lambda b: (b, 0, 0)),
            pl.BlockSpec((1, 1, _ZD), lambda b: (b, 0, 0)),
        ),
        out_shape=out_shapes,
        compiler_params=cparams2,
    )(r_sum, eps3, x_target,
      Wh.astype(bf16), bh.reshape(1, _HD),
      Wmu.astype(bf16), bmu.reshape(1, _ZD),
      Wsig.astype(bf16), bsig.reshape(1, _ZD),
      Wd1[0:1, :].astype(bf16), Wd1[1:, :].astype(bf16), bd1.reshape(1, _HD),
      Wd2.astype(bf16), bd2.reshape(1, _HD),
      Wdmu.astype(bf16), bdmu.reshape(1, YD),
      Wdsig.astype(bf16), bdsig.reshape(1, YD))

    return y_mu, y_sigma, mu_c.reshape(B, _ZD), sigma_c.reshape(B, _ZD)


# speedup vs baseline: 1.2413x; 1.2413x over previous
import jax
import jax.numpy as jnp
from jax.experimental import pallas as pl
from jax.experimental.pallas import tpu as pltpu

_G = 50
_GP = _G + 2
_P = _GP * _GP
_PL = 2816
_EXT = _P + 106
_OFFS = (0, 1, 2, _GP, _GP + 1, _GP + 2, 2 * _GP, 2 * _GP + 1, 2 * _GP + 2)
_CC = 32
_RD = 128
_ZD = 64
_HD = 128


_S = 4


def _enc_body(yx_ref, me_ref, W1y_ref, W1x_ref, b1_ref, W2_ref, b2_ref,
              We1_ref, be1_ref, We2_ref, be2_ref, out_ref):
    n = pl.program_id(1)
    bf16 = jnp.bfloat16
    me = me_ref[...]
    maskt = me[:, 53:53 + _P]

    O9 = jnp.concatenate(
        [me[:, s:s + _P] for s in _OFFS], axis=0).astype(bf16)
    X1m = jnp.dot(W1x_ref[...], O9,
                  preferred_element_type=jnp.float32)

    feats = []
    for i in range(_S):
        row = yx_ref[0, i]
        yext = row[:, :_EXT]
        x11 = row[:, _EXT:_EXT + 1]

        Y9 = jnp.concatenate(
            [yext[:, s:s + _P] for s in _OFFS], axis=0).astype(bf16)
        H1 = jnp.dot(W1y_ref[...], Y9, preferred_element_type=jnp.float32)
        H1 = jnp.maximum(H1 + x11 * X1m + b1_ref[...], 0.0)
        H1 = H1 * maskt

        z53 = jnp.zeros((_CC, 53), jnp.float32)
        H1e = jnp.concatenate([z53, H1, z53], axis=1)
        C2 = jnp.concatenate(
            [H1e[:, s:s + _P] for s in _OFFS], axis=0).astype(bf16)
        H2 = jnp.dot(W2_ref[...], C2, preferred_element_type=jnp.float32)
        H2 = jnp.maximum(H2 + b2_ref[...], 0.0)

        pooled = jax.lax.dot_general(
            maskt.astype(bf16), H2.astype(bf16), (((1,), (1,)), ((), ())),
            preferred_element_type=jnp.float32) * (1.0 / 2500.0)
        feats.append(jnp.concatenate([pooled, x11], axis=1))

    feat = jnp.concatenate(feats, axis=0).astype(bf16)
    h = jnp.maximum(jnp.dot(feat, We1_ref[...],
                            preferred_element_type=jnp.float32)
                    + be1_ref[...], 0.0)
    r_i = jnp.dot(h.astype(bf16), We2_ref[...],
                  preferred_element_type=jnp.float32) + be2_ref[...]
    r_p = jnp.sum(r_i, axis=0, keepdims=True)

    @pl.when(n == 0)
    def _():
        out_ref[0] = r_p

    @pl.when(n != 0)
    def _():
        out_ref[0] = out_ref[0] + r_p


def _dec_body(rs_ref, eps_ref, xt_ref,
              Wh_ref, bh_ref, Wmu_ref, bmu_ref, Wsig_ref, bsig_ref,
              W1x_ref, W1z_ref, bd1_ref, Wd2_ref, bd2_ref,
              Wdmu_ref, bdmu_ref, Wdsig_ref, bdsig_ref,
              ymu_ref, ysig_ref, mu_ref, sig_ref):
    bf16 = jnp.bfloat16
    r = rs_ref[0] * (1.0 / 64.0)
    hr = jnp.maximum(jnp.dot(r.astype(bf16), Wh_ref[...],
                             preferred_element_type=jnp.float32)
                     + bh_ref[...], 0.0).astype(bf16)
    mu = jnp.dot(hr, Wmu_ref[...],
                 preferred_element_type=jnp.float32) + bmu_ref[...]
    sg = 0.1 + 0.9 * jax.nn.sigmoid(
        jnp.dot(hr, Wsig_ref[...],
                preferred_element_type=jnp.float32) + bsig_ref[...])
    z = mu + sg * eps_ref[0]

    xt = xt_ref[0]
    h1 = jnp.dot(xt.astype(bf16), W1x_ref[...],
                 preferred_element_type=jnp.float32)
    h1 = h1 + jnp.dot(z.astype(bf16), W1z_ref[...],
                      preferred_element_type=jnp.float32) + bd1_ref[...]
    h1 = jnp.maximum(h1, 0.0).astype(bf16)
    h2 = jnp.maximum(jnp.dot(h1, Wd2_ref[...],
                             preferred_element_type=jnp.float32)
                     + bd2_ref[...], 0.0).astype(bf16)
    ymu_ref[0] = jnp.dot(h2, Wdmu_ref[...],
                         preferred_element_type=jnp.float32) + bdmu_ref[...]
    t = jnp.dot(h2, Wdsig_ref[...],
                preferred_element_type=jnp.float32) + bdsig_ref[...]
    sp = jnp.maximum(t, 0.0) + jnp.log1p(jnp.exp(-jnp.abs(t)))
    ysig_ref[0] = 0.1 + 0.9 * sp
    mu_ref[0] = mu
    sig_ref[0] = sg


def kernel(x_context, y_context, x_target, eps,
           w1, b1, w2, b2, We1, be1, We2, be2,
           Wh, bh, Wmu, bmu, Wsig, bsig,
           Wd1, bd1, Wd2, bd2, Wdmu, bdmu, Wdsig, bdsig):
    f32 = jnp.float32
    B, NC, _ = x_context.shape
    NT = x_target.shape[1]
    YD = y_context.shape[2]

    yg = y_context.reshape(B, NC, _G, _G)
    ygp = jnp.pad(yg, ((0, 0), (0, 0), (1, 1), (1, 1)))
    yflat = ygp.reshape(B, NC, _P)
    yext = jnp.pad(yflat, ((0, 0), (0, 0), (53, 53)))
    yx = jnp.concatenate([yext, x_context], axis=2)
    yx = jnp.pad(yx, ((0, 0), (0, 0), (0, _PL - _EXT - 1)))
    yx = yx.reshape(B, NC, 1, _PL)

    m2 = jnp.pad(jnp.ones((_G, _G), f32), ((1, 1), (1, 1))).reshape(1, _P)
    maskext = jnp.pad(m2, ((0, 0), (53, _PL - _P - 53)))

    bf16 = jnp.bfloat16
    W1y = w1[:, 0].reshape(_CC, 9).astype(bf16)
    W1x = w1[:, 1].reshape(_CC, 9).astype(bf16)
    b1c = b1.reshape(_CC, 1)
    W2T = jnp.transpose(w2, (0, 2, 3, 1)).reshape(_CC, 9 * _CC).astype(bf16)
    b2c = b2.reshape(_CC, 1)
    be1r = be1.reshape(1, _HD)
    be2r = be2.reshape(1, _RD)

    cparams = pltpu.CompilerParams(
        dimension_semantics=("parallel", "arbitrary"),
        vmem_limit_bytes=100 * 1024 * 1024,
    )
    full = lambda shp: pl.BlockSpec(shp, lambda b, n: (0,) * len(shp))
    r_sum = pl.pallas_call(
        _enc_body,
        grid=(B, NC // _S),
        in_specs=[
            pl.BlockSpec((1, _S, 1, _PL), lambda b, n: (b, n, 0, 0)),
            full((1, _PL)),
            full((_CC, 9)), full((_CC, 9)), full((_CC, 1)),
            full((_CC, 9 * _CC)), full((_CC, 1)),
            full((_CC + 1, _HD)), full((1, _HD)),
            full((_HD, _RD)), full((1, _RD)),
        ],
        out_specs=pl.BlockSpec((1, 1, _RD), lambda b, n: (b, 0, 0)),
        out_shape=jax.ShapeDtypeStruct((B, 1, _RD), f32),
        compiler_params=cparams,
    )(yx, maskext, W1y, W1x, b1c, W2T, b2c,
      We1.astype(bf16), be1r, We2.astype(bf16), be2r)

    cparams2 = pltpu.CompilerParams(
        dimension_semantics=("parallel",),
        vmem_limit_bytes=100 * 1024 * 1024,
    )
    full1 = lambda shp: pl.BlockSpec(shp, lambda b: (0,) * len(shp))
    eps3 = eps.reshape(B, 1, _ZD)
    out_shapes = (
        jax.ShapeDtypeStruct((B, NT, YD), f32),
        jax.ShapeDtypeStruct((B, NT, YD), f32),
        jax.ShapeDtypeStruct((B, 1, _ZD), f32),
        jax.ShapeDtypeStruct((B, 1, _ZD), f32),
    )
    y_mu, y_sigma, mu_c, sigma_c = pl.pallas_call(
        _dec_body,
        grid=(B,),
        in_specs=[
            pl.BlockSpec((1, 1, _RD), lambda b: (b, 0, 0)),
            pl.BlockSpec((1, 1, _ZD), lambda b: (b, 0, 0)),
            pl.BlockSpec((1, NT, 1), lambda b: (b, 0, 0)),
            full1((_RD, _HD)), full1((1, _HD)),
            full1((_HD, _ZD)), full1((1, _ZD)),
            full1((_HD, _ZD)), full1((1, _ZD)),
            full1((1, _HD)), full1((_ZD, _HD)), full1((1, _HD)),
            full1((_HD, _HD)), full1((1, _HD)),
            full1((_HD, YD)), full1((1, YD)),
            full1((_HD, YD)), full1((1, YD)),
        ],
        out_specs=(
            pl.BlockSpec((1, NT, YD), lambda b: (b, 0, 0)),
            pl.BlockSpec((1, NT, YD), lambda b: (b, 0, 0)),
            pl.BlockSpec((1, 1, _ZD), lambda b: (b, 0, 0)),
            pl.BlockSpec((1, 1, _ZD), lambda b: (b, 0, 0)),
        ),
        out_shape=out_shapes,
        compiler_params=cparams2,
    )(r_sum, eps3, x_target,
      Wh.astype(bf16), bh.reshape(1, _HD),
      Wmu.astype(bf16), bmu.reshape(1, _ZD),
      Wsig.astype(bf16), bsig.reshape(1, _ZD),
      Wd1[0:1, :].astype(bf16), Wd1[1:, :].astype(bf16), bd1.reshape(1, _HD),
      Wd2.astype(bf16), bd2.reshape(1, _HD),
      Wdmu.astype(bf16), bdmu.reshape(1, YD),
      Wdsig.astype(bf16), bdsig.reshape(1, YD))

    return y_mu, y_sigma, mu_c.reshape(B, _ZD), sigma_c.reshape(B, _ZD)


# speedup vs baseline: 1.2642x; 1.0184x over previous
import jax
import jax.numpy as jnp
from jax.experimental import pallas as pl
from jax.experimental.pallas import tpu as pltpu

_G = 50
_GP = _G + 2
_P = _GP * _GP
_PL = 2816
_EXT = _P + 106
_OFFS = (0, 1, 2, _GP, _GP + 1, _GP + 2, 2 * _GP, 2 * _GP + 1, 2 * _GP + 2)
_CC = 32
_RD = 128
_ZD = 64
_HD = 128


_S = 8


def _enc_body(yx_ref, me_ref, W1y_ref, W1x_ref, b1_ref, W2_ref, b2_ref,
              We1_ref, be1_ref, We2_ref, be2_ref, out_ref):
    n = pl.program_id(1)
    bf16 = jnp.bfloat16
    me = me_ref[...]
    maskt = me[:, 53:53 + _P]

    O9 = jnp.concatenate(
        [me[:, s:s + _P] for s in _OFFS], axis=0).astype(bf16)
    X1m = jnp.dot(W1x_ref[...], O9,
                  preferred_element_type=jnp.float32)

    feats = []
    for i in range(_S):
        row = yx_ref[0, i]
        yext = row[:, :_EXT]
        x11 = row[:, _EXT:_EXT + 1]

        Y9 = jnp.concatenate(
            [yext[:, s:s + _P] for s in _OFFS], axis=0).astype(bf16)
        H1 = jnp.dot(W1y_ref[...], Y9, preferred_element_type=jnp.float32)
        H1 = jnp.maximum(H1 + x11 * X1m + b1_ref[...], 0.0)
        H1 = H1 * maskt

        z53 = jnp.zeros((_CC, 53), jnp.float32)
        H1e = jnp.concatenate([z53, H1, z53], axis=1).astype(bf16)
        G = jnp.dot(W2_ref[...], H1e,
                    preferred_element_type=jnp.float32).astype(bf16)
        H2 = G[0:_CC, 0:_P].astype(jnp.float32)
        for k in range(1, 9):
            s = _OFFS[k]
            H2 = H2 + G[k * _CC:(k + 1) * _CC, s:s + _P].astype(jnp.float32)
        H2 = jnp.maximum(H2 + b2_ref[...], 0.0)

        pooled = jax.lax.dot_general(
            maskt.astype(bf16), H2.astype(bf16), (((1,), (1,)), ((), ())),
            preferred_element_type=jnp.float32) * (1.0 / 2500.0)
        feats.append(jnp.concatenate([pooled, x11], axis=1))

    feat = jnp.concatenate(feats, axis=0).astype(bf16)
    h = jnp.maximum(jnp.dot(feat, We1_ref[...],
                            preferred_element_type=jnp.float32)
                    + be1_ref[...], 0.0)
    r_i = jnp.dot(h.astype(bf16), We2_ref[...],
                  preferred_element_type=jnp.float32) + be2_ref[...]
    r_p = jnp.sum(r_i, axis=0, keepdims=True)

    @pl.when(n == 0)
    def _():
        out_ref[0] = r_p

    @pl.when(n != 0)
    def _():
        out_ref[0] = out_ref[0] + r_p


def _dec_body(rs_ref, eps_ref, xt_ref,
              Wh_ref, bh_ref, Wmu_ref, bmu_ref, Wsig_ref, bsig_ref,
              W1x_ref, W1z_ref, bd1_ref, Wd2_ref, bd2_ref,
              Wdmu_ref, bdmu_ref, Wdsig_ref, bdsig_ref,
              ymu_ref, ysig_ref, mu_ref, sig_ref):
    bf16 = jnp.bfloat16
    r = rs_ref[0] * (1.0 / 64.0)
    hr = jnp.maximum(jnp.dot(r.astype(bf16), Wh_ref[...],
                             preferred_element_type=jnp.float32)
                     + bh_ref[...], 0.0).astype(bf16)
    mu = jnp.dot(hr, Wmu_ref[...],
                 preferred_element_type=jnp.float32) + bmu_ref[...]
    sg = 0.1 + 0.9 * jax.nn.sigmoid(
        jnp.dot(hr, Wsig_ref[...],
                preferred_element_type=jnp.float32) + bsig_ref[...])
    z = mu + sg * eps_ref[0]

    xt = xt_ref[0]
    h1 = jnp.dot(xt.astype(bf16), W1x_ref[...],
                 preferred_element_type=jnp.float32)
    h1 = h1 + jnp.dot(z.astype(bf16), W1z_ref[...],
                      preferred_element_type=jnp.float32) + bd1_ref[...]
    h1 = jnp.maximum(h1, 0.0).astype(bf16)
    h2 = jnp.maximum(jnp.dot(h1, Wd2_ref[...],
                             preferred_element_type=jnp.float32)
                     + bd2_ref[...], 0.0).astype(bf16)
    ymu_ref[0] = jnp.dot(h2, Wdmu_ref[...],
                         preferred_element_type=jnp.float32) + bdmu_ref[...]
    t = jnp.dot(h2, Wdsig_ref[...],
                preferred_element_type=jnp.float32) + bdsig_ref[...]
    sp = jnp.maximum(t, 0.0) + jnp.log1p(jnp.exp(-jnp.abs(t)))
    ysig_ref[0] = 0.1 + 0.9 * sp
    mu_ref[0] = mu
    sig_ref[0] = sg


def kernel(x_context, y_context, x_target, eps,
           w1, b1, w2, b2, We1, be1, We2, be2,
           Wh, bh, Wmu, bmu, Wsig, bsig,
           Wd1, bd1, Wd2, bd2, Wdmu, bdmu, Wdsig, bdsig):
    f32 = jnp.float32
    B, NC, _ = x_context.shape
    NT = x_target.shape[1]
    YD = y_context.shape[2]

    yg = y_context.reshape(B, NC, _G, _G)
    ygp = jnp.pad(yg, ((0, 0), (0, 0), (1, 1), (1, 1)))
    yflat = ygp.reshape(B, NC, _P)
    yext = jnp.pad(yflat, ((0, 0), (0, 0), (53, 53)))
    yx = jnp.concatenate([yext, x_context], axis=2)
    yx = jnp.pad(yx, ((0, 0), (0, 0), (0, _PL - _EXT - 1)))
    yx = yx.reshape(B, NC, 1, _PL)

    m2 = jnp.pad(jnp.ones((_G, _G), f32), ((1, 1), (1, 1))).reshape(1, _P)
    maskext = jnp.pad(m2, ((0, 0), (53, _PL - _P - 53)))

    bf16 = jnp.bfloat16
    W1y = w1[:, 0].reshape(_CC, 9).astype(bf16)
    W1x = w1[:, 1].reshape(_CC, 9).astype(bf16)
    b1c = b1.reshape(_CC, 1)
    W2T = jnp.transpose(w2, (2, 3, 0, 1)).reshape(9 * _CC, _CC).astype(bf16)
    b2c = b2.reshape(_CC, 1)
    be1r = be1.reshape(1, _HD)
    be2r = be2.reshape(1, _RD)

    cparams = pltpu.CompilerParams(
        dimension_semantics=("parallel", "arbitrary"),
        vmem_limit_bytes=100 * 1024 * 1024,
    )
    full = lambda shp: pl.BlockSpec(shp, lambda b, n: (0,) * len(shp))
    r_sum = pl.pallas_call(
        _enc_body,
        grid=(B, NC // _S),
        in_specs=[
            pl.BlockSpec((1, _S, 1, _PL), lambda b, n: (b, n, 0, 0)),
            full((1, _PL)),
            full((_CC, 9)), full((_CC, 9)), full((_CC, 1)),
            full((9 * _CC, _CC)), full((_CC, 1)),
            full((_CC + 1, _HD)), full((1, _HD)),
            full((_HD, _RD)), full((1, _RD)),
        ],
        out_specs=pl.BlockSpec((1, 1, _RD), lambda b, n: (b, 0, 0)),
        out_shape=jax.ShapeDtypeStruct((B, 1, _RD), f32),
        compiler_params=cparams,
    )(yx, maskext, W1y, W1x, b1c, W2T, b2c,
      We1.astype(bf16), be1r, We2.astype(bf16), be2r)

    cparams2 = pltpu.CompilerParams(
        dimension_semantics=("parallel",),
        vmem_limit_bytes=100 * 1024 * 1024,
    )
    full1 = lambda shp: pl.BlockSpec(shp, lambda b: (0,) * len(shp))
    eps3 = eps.reshape(B, 1, _ZD)
    out_shapes = (
        jax.ShapeDtypeStruct((B, NT, YD), f32),
        jax.ShapeDtypeStruct((B, NT, YD), f32),
        jax.ShapeDtypeStruct((B, 1, _ZD), f32),
        jax.ShapeDtypeStruct((B, 1, _ZD), f32),
    )
    y_mu, y_sigma, mu_c, sigma_c = pl.pallas_call(
        _dec_body,
        grid=(B,),
        in_specs=[
            pl.BlockSpec((1, 1, _RD), lambda b: (b, 0, 0)),
            pl.BlockSpec((1, 1, _ZD), lambda b: (b, 0, 0)),
            pl.BlockSpec((1, NT, 1), lambda b: (b, 0, 0)),
            full1((_RD, _HD)), full1((1, _HD)),
            full1((_HD, _ZD)), full1((1, _ZD)),
            full1((_HD, _ZD)), full1((1, _ZD)),
            full1((1, _HD)), full1((_ZD, _HD)), full1((1, _HD)),
            full1((_HD, _HD)), full1((1, _HD)),
            full1((_HD, YD)), full1((1, YD)),
            full1((_HD, YD)), full1((1, YD)),
        ],
        out_specs=(
            pl.BlockSpec((1, NT, YD), lambda b: (b, 0, 0)),
            pl.BlockSpec((1, NT, YD), lambda b: (b, 0, 0)),
            pl.BlockSpec((1, 1, _ZD), lambda b: (b, 0, 0)),
            pl.BlockSpec((1, 1, _ZD), lambda b: (b, 0, 0)),
        ),
        out_shape=out_shapes,
        compiler_params=cparams2,
    )(r_sum, eps3, x_target,
      Wh.astype(bf16), bh.reshape(1, _HD),
      Wmu.astype(bf16), bmu.reshape(1, _ZD),
      Wsig.astype(bf16), bsig.reshape(1, _ZD),
      Wd1[0:1, :].astype(bf16), Wd1[1:, :].astype(bf16), bd1.reshape(1, _HD),
      Wd2.astype(bf16), bd2.reshape(1, _HD),
      Wdmu.astype(bf16), bdmu.reshape(1, YD),
      Wdsig.astype(bf16), bdsig.reshape(1, YD))

    return y_mu, y_sigma, mu_c.reshape(B, _ZD), sigma_c.reshape(B, _ZD)


# speedup vs baseline: 1.2831x; 1.0150x over previous
import jax
import jax.numpy as jnp
from jax.experimental import pallas as pl
from jax.experimental.pallas import tpu as pltpu

_G = 50
_GP = _G + 2
_P = _GP * _GP
_PL = 2816
_EXT = _P + 106
_OFFS = (0, 1, 2, _GP, _GP + 1, _GP + 2, 2 * _GP, 2 * _GP + 1, 2 * _GP + 2)
_CC = 32
_RD = 128
_ZD = 64
_HD = 128


_S = 16


def _enc_body(yx_ref, me_ref, W1y_ref, W1x_ref, b1_ref, W2_ref, b2_ref,
              We1_ref, be1_ref, We2_ref, be2_ref, out_ref):
    n = pl.program_id(1)
    bf16 = jnp.bfloat16
    me = me_ref[...]
    maskt = me[:, 53:53 + _P]

    O9 = jnp.concatenate(
        [me[:, s:s + _P] for s in _OFFS], axis=0).astype(bf16)
    X1m = jnp.dot(W1x_ref[...], O9,
                  preferred_element_type=jnp.float32)

    feats = []
    for i in range(_S):
        row = yx_ref[0, i]
        yext = row[:, :_EXT]
        x11 = row[:, _EXT:_EXT + 1]

        Y9 = jnp.concatenate(
            [yext[:, s:s + _P] for s in _OFFS], axis=0).astype(bf16)
        H1 = jnp.dot(W1y_ref[...], Y9, preferred_element_type=jnp.float32)
        H1 = jnp.maximum(H1 + x11 * X1m + b1_ref[...], 0.0)
        H1 = H1 * maskt

        z53 = jnp.zeros((_CC, 53), jnp.float32)
        H1e = jnp.concatenate([z53, H1, z53], axis=1).astype(bf16)
        G = jnp.dot(W2_ref[...], H1e,
                    preferred_element_type=jnp.float32).astype(bf16)
        H2 = G[0:_CC, 0:_P].astype(jnp.float32)
        for k in range(1, 9):
            s = _OFFS[k]
            H2 = H2 + G[k * _CC:(k + 1) * _CC, s:s + _P].astype(jnp.float32)
        H2 = jnp.maximum(H2 + b2_ref[...], 0.0)

        pooled = jax.lax.dot_general(
            maskt.astype(bf16), H2.astype(bf16), (((1,), (1,)), ((), ())),
            preferred_element_type=jnp.float32) * (1.0 / 2500.0)
        feats.append(jnp.concatenate([pooled, x11], axis=1))

    feat = jnp.concatenate(feats, axis=0).astype(bf16)
    h = jnp.maximum(jnp.dot(feat, We1_ref[...],
                            preferred_element_type=jnp.float32)
                    + be1_ref[...], 0.0)
    r_i = jnp.dot(h.astype(bf16), We2_ref[...],
                  preferred_element_type=jnp.float32) + be2_ref[...]
    r_p = jnp.sum(r_i, axis=0, keepdims=True)

    @pl.when(n == 0)
    def _():
        out_ref[0] = r_p

    @pl.when(n != 0)
    def _():
        out_ref[0] = out_ref[0] + r_p


def _dec_body(rs_ref, eps_ref, xt_ref,
              Wh_ref, bh_ref, Wmu_ref, bmu_ref, Wsig_ref, bsig_ref,
              W1x_ref, W1z_ref, bd1_ref, Wd2_ref, bd2_ref,
              Wdmu_ref, bdmu_ref, Wdsig_ref, bdsig_ref,
              ymu_ref, ysig_ref, mu_ref, sig_ref):
    bf16 = jnp.bfloat16
    r = rs_ref[0] * (1.0 / 64.0)
    hr = jnp.maximum(jnp.dot(r.astype(bf16), Wh_ref[...],
                             preferred_element_type=jnp.float32)
                     + bh_ref[...], 0.0).astype(bf16)
    mu = jnp.dot(hr, Wmu_ref[...],
                 preferred_element_type=jnp.float32) + bmu_ref[...]
    sg = 0.1 + 0.9 * jax.nn.sigmoid(
        jnp.dot(hr, Wsig_ref[...],
                preferred_element_type=jnp.float32) + bsig_ref[...])
    z = mu + sg * eps_ref[0]

    xt = xt_ref[0]
    h1 = jnp.dot(xt.astype(bf16), W1x_ref[...],
                 preferred_element_type=jnp.float32)
    h1 = h1 + jnp.dot(z.astype(bf16), W1z_ref[...],
                      preferred_element_type=jnp.float32) + bd1_ref[...]
    h1 = jnp.maximum(h1, 0.0).astype(bf16)
    h2 = jnp.maximum(jnp.dot(h1, Wd2_ref[...],
                             preferred_element_type=jnp.float32)
                     + bd2_ref[...], 0.0).astype(bf16)
    ymu_ref[0] = jnp.dot(h2, Wdmu_ref[...],
                         preferred_element_type=jnp.float32) + bdmu_ref[...]
    t = jnp.dot(h2, Wdsig_ref[...],
                preferred_element_type=jnp.float32) + bdsig_ref[...]
    sp = jnp.maximum(t, 0.0) + jnp.log1p(jnp.exp(-jnp.abs(t)))
    ysig_ref[0] = 0.1 + 0.9 * sp
    mu_ref[0] = mu
    sig_ref[0] = sg


def kernel(x_context, y_context, x_target, eps,
           w1, b1, w2, b2, We1, be1, We2, be2,
           Wh, bh, Wmu, bmu, Wsig, bsig,
           Wd1, bd1, Wd2, bd2, Wdmu, bdmu, Wdsig, bdsig):
    f32 = jnp.float32
    B, NC, _ = x_context.shape
    NT = x_target.shape[1]
    YD = y_context.shape[2]

    yg = y_context.reshape(B, NC, _G, _G)
    ygp = jnp.pad(yg, ((0, 0), (0, 0), (1, 1), (1, 1)))
    yflat = ygp.reshape(B, NC, _P)
    yext = jnp.pad(yflat, ((0, 0), (0, 0), (53, 53)))
    yx = jnp.concatenate([yext, x_context], axis=2)
    yx = jnp.pad(yx, ((0, 0), (0, 0), (0, _PL - _EXT - 1)))
    yx = yx.reshape(B, NC, 1, _PL)

    m2 = jnp.pad(jnp.ones((_G, _G), f32), ((1, 1), (1, 1))).reshape(1, _P)
    maskext = jnp.pad(m2, ((0, 0), (53, _PL - _P - 53)))

    bf16 = jnp.bfloat16
    W1y = w1[:, 0].reshape(_CC, 9).astype(bf16)
    W1x = w1[:, 1].reshape(_CC, 9).astype(bf16)
    b1c = b1.reshape(_CC, 1)
    W2T = jnp.transpose(w2, (2, 3, 0, 1)).reshape(9 * _CC, _CC).astype(bf16)
    b2c = b2.reshape(_CC, 1)
    be1r = be1.reshape(1, _HD)
    be2r = be2.reshape(1, _RD)

    cparams = pltpu.CompilerParams(
        dimension_semantics=("parallel", "arbitrary"),
        vmem_limit_bytes=100 * 1024 * 1024,
    )
    full = lambda shp: pl.BlockSpec(shp, lambda b, n: (0,) * len(shp))
    r_sum = pl.pallas_call(
        _enc_body,
        grid=(B, NC // _S),
        in_specs=[
            pl.BlockSpec((1, _S, 1, _PL), lambda b, n: (b, n, 0, 0)),
            full((1, _PL)),
            full((_CC, 9)), full((_CC, 9)), full((_CC, 1)),
            full((9 * _CC, _CC)), full((_CC, 1)),
            full((_CC + 1, _HD)), full((1, _HD)),
            full((_HD, _RD)), full((1, _RD)),
        ],
        out_specs=pl.BlockSpec((1, 1, _RD), lambda b, n: (b, 0, 0)),
        out_shape=jax.ShapeDtypeStruct((B, 1, _RD), f32),
        compiler_params=cparams,
    )(yx, maskext, W1y, W1x, b1c, W2T, b2c,
      We1.astype(bf16), be1r, We2.astype(bf16), be2r)

    cparams2 = pltpu.CompilerParams(
        dimension_semantics=("parallel",),
        vmem_limit_bytes=100 * 1024 * 1024,
    )
    full1 = lambda shp: pl.BlockSpec(shp, lambda b: (0,) * len(shp))
    eps3 = eps.reshape(B, 1, _ZD)
    out_shapes = (
        jax.ShapeDtypeStruct((B, NT, YD), f32),
        jax.ShapeDtypeStruct((B, NT, YD), f32),
        jax.ShapeDtypeStruct((B, 1, _ZD), f32),
        jax.ShapeDtypeStruct((B, 1, _ZD), f32),
    )
    y_mu, y_sigma, mu_c, sigma_c = pl.pallas_call(
        _dec_body,
        grid=(B,),
        in_specs=[
            pl.BlockSpec((1, 1, _RD), lambda b: (b, 0, 0)),
            pl.BlockSpec((1, 1, _ZD), lambda b: (b, 0, 0)),
            pl.BlockSpec((1, NT, 1), lambda b: (b, 0, 0)),
            full1((_RD, _HD)), full1((1, _HD)),
            full1((_HD, _ZD)), full1((1, _ZD)),
            full1((_HD, _ZD)), full1((1, _ZD)),
            full1((1, _HD)), full1((_ZD, _HD)), full1((1, _HD)),
            full1((_HD, _HD)), full1((1, _HD)),
            full1((_HD, YD)), full1((1, YD)),
            full1((_HD, YD)), full1((1, YD)),
        ],
        out_specs=(
            pl.BlockSpec((1, NT, YD), lambda b: (b, 0, 0)),
            pl.BlockSpec((1, NT, YD), lambda b: (b, 0, 0)),
            pl.BlockSpec((1, 1, _ZD), lambda b: (b, 0, 0)),
            pl.BlockSpec((1, 1, _ZD), lambda b: (b, 0, 0)),
        ),
        out_shape=out_shapes,
        compiler_params=cparams2,
    )(r_sum, eps3, x_target,
      Wh.astype(bf16), bh.reshape(1, _HD),
      Wmu.astype(bf16), bmu.reshape(1, _ZD),
      Wsig.astype(bf16), bsig.reshape(1, _ZD),
      Wd1[0:1, :].astype(bf16), Wd1[1:, :].astype(bf16), bd1.reshape(1, _HD),
      Wd2.astype(bf16), bd2.reshape(1, _HD),
      Wdmu.astype(bf16), bdmu.reshape(1, YD),
      Wdsig.astype(bf16), bdsig.reshape(1, YD))

    return y_mu, y_sigma, mu_c.reshape(B, _ZD), sigma_c.reshape(B, _ZD)


# speedup vs baseline: 1.2928x; 1.0075x over previous
import jax
import jax.numpy as jnp
from jax.experimental import pallas as pl
from jax.experimental.pallas import tpu as pltpu

_G = 50
_GP = _G + 2
_P = _GP * _GP
_PL = 2816
_EXT = _P + 106
_OFFS = (0, 1, 2, _GP, _GP + 1, _GP + 2, 2 * _GP, 2 * _GP + 1, 2 * _GP + 2)
_CC = 32
_RD = 128
_ZD = 64
_HD = 128


_S = 16


def _enc_body(yx_ref, x_ref, me_ref, W1y_ref, W1x_ref, b1_ref, W2_ref, b2_ref,
              We1_ref, be1_ref, We2_ref, be2_ref, out_ref):
    n = pl.program_id(1)
    bf16 = jnp.bfloat16
    me = me_ref[...]
    maskt = me[:, 53:53 + _P]

    O9 = jnp.concatenate(
        [me[:, s:s + _P] for s in _OFFS], axis=0).astype(bf16)
    X1m = jnp.dot(W1x_ref[...], O9,
                  preferred_element_type=jnp.float32)

    feats = []
    for i in range(_S):
        row = yx_ref[0, i]
        yext = row[:, :_EXT]
        x11 = x_ref[0, 0, :, i:i + 1]

        Y9 = jnp.concatenate(
            [yext[:, s:s + _P] for s in _OFFS], axis=0).astype(bf16)
        H1 = jnp.dot(W1y_ref[...], Y9, preferred_element_type=jnp.float32)
        H1 = jnp.maximum(H1 + x11 * X1m + b1_ref[...], 0.0)
        H1 = H1 * maskt

        z53 = jnp.zeros((_CC, 53), jnp.float32)
        H1e = jnp.concatenate([z53, H1, z53], axis=1).astype(bf16)
        G = jnp.dot(W2_ref[...], H1e,
                    preferred_element_type=jnp.float32).astype(bf16)
        H2 = G[0:_CC, 0:_P].astype(jnp.float32)
        for k in range(1, 9):
            s = _OFFS[k]
            H2 = H2 + G[k * _CC:(k + 1) * _CC, s:s + _P].astype(jnp.float32)
        H2 = jnp.maximum(H2 + b2_ref[...], 0.0)

        pooled = jax.lax.dot_general(
            maskt.astype(bf16), H2.astype(bf16), (((1,), (1,)), ((), ())),
            preferred_element_type=jnp.float32) * (1.0 / 2500.0)
        feats.append(jnp.concatenate([pooled, x11], axis=1))

    feat = jnp.concatenate(feats, axis=0).astype(bf16)
    h = jnp.maximum(jnp.dot(feat, We1_ref[...],
                            preferred_element_type=jnp.float32)
                    + be1_ref[...], 0.0)
    r_i = jnp.dot(h.astype(bf16), We2_ref[...],
                  preferred_element_type=jnp.float32) + be2_ref[...]
    r_p = jnp.sum(r_i, axis=0, keepdims=True)

    @pl.when(n == 0)
    def _():
        out_ref[0] = r_p

    @pl.when(n != 0)
    def _():
        out_ref[0] = out_ref[0] + r_p


def _dec_body(rs_ref, eps_ref, xt_ref,
              Wh_ref, bh_ref, Wmu_ref, bmu_ref, Wsig_ref, bsig_ref,
              W1x_ref, W1z_ref, bd1_ref, Wd2_ref, bd2_ref,
              Wdmu_ref, bdmu_ref, Wdsig_ref, bdsig_ref,
              ymu_ref, ysig_ref, mu_ref, sig_ref):
    bf16 = jnp.bfloat16
    r = rs_ref[0] * (1.0 / 64.0)
    hr = jnp.maximum(jnp.dot(r.astype(bf16), Wh_ref[...],
                             preferred_element_type=jnp.float32)
                     + bh_ref[...], 0.0).astype(bf16)
    mu = jnp.dot(hr, Wmu_ref[...],
                 preferred_element_type=jnp.float32) + bmu_ref[...]
    sg = 0.1 + 0.9 * jax.nn.sigmoid(
        jnp.dot(hr, Wsig_ref[...],
                preferred_element_type=jnp.float32) + bsig_ref[...])
    z = mu + sg * eps_ref[0]

    xt = xt_ref[0]
    h1 = jnp.dot(xt.astype(bf16), W1x_ref[...],
                 preferred_element_type=jnp.float32)
    h1 = h1 + jnp.dot(z.astype(bf16), W1z_ref[...],
                      preferred_element_type=jnp.float32) + bd1_ref[...]
    h1 = jnp.maximum(h1, 0.0).astype(bf16)
    h2 = jnp.maximum(jnp.dot(h1, Wd2_ref[...],
                             preferred_element_type=jnp.float32)
                     + bd2_ref[...], 0.0).astype(bf16)
    ymu_ref[0] = jnp.dot(h2, Wdmu_ref[...],
                         preferred_element_type=jnp.float32) + bdmu_ref[...]
    t = jnp.dot(h2, Wdsig_ref[...],
                preferred_element_type=jnp.float32) + bdsig_ref[...]
    sp = jnp.maximum(t, 0.0) + jnp.log1p(jnp.exp(-jnp.abs(t)))
    ysig_ref[0] = 0.1 + 0.9 * sp
    mu_ref[0] = mu
    sig_ref[0] = sg


def kernel(x_context, y_context, x_target, eps,
           w1, b1, w2, b2, We1, be1, We2, be2,
           Wh, bh, Wmu, bmu, Wsig, bsig,
           Wd1, bd1, Wd2, bd2, Wdmu, bdmu, Wdsig, bdsig):
    f32 = jnp.float32
    B, NC, _ = x_context.shape
    NT = x_target.shape[1]
    YD = y_context.shape[2]

    yg = y_context.reshape(B, NC, _G, _G)
    ygp = jnp.pad(yg, ((0, 0), (0, 0), (1, 1), (1, 1)))
    yflat = ygp.reshape(B, NC, _P)
    yx = jnp.pad(yflat, ((0, 0), (0, 0), (53, _PL - _P - 53)))
    yx = yx.reshape(B, NC, 1, _PL)
    xs = x_context.reshape(B, NC // _S, 1, _S)

    m2 = jnp.pad(jnp.ones((_G, _G), f32), ((1, 1), (1, 1))).reshape(1, _P)
    maskext = jnp.pad(m2, ((0, 0), (53, _PL - _P - 53)))

    bf16 = jnp.bfloat16
    W1y = w1[:, 0].reshape(_CC, 9).astype(bf16)
    W1x = w1[:, 1].reshape(_CC, 9).astype(bf16)
    b1c = b1.reshape(_CC, 1)
    W2T = jnp.transpose(w2, (2, 3, 0, 1)).reshape(9 * _CC, _CC).astype(bf16)
    b2c = b2.reshape(_CC, 1)
    be1r = be1.reshape(1, _HD)
    be2r = be2.reshape(1, _RD)

    cparams = pltpu.CompilerParams(
        dimension_semantics=("parallel", "arbitrary"),
        vmem_limit_bytes=100 * 1024 * 1024,
    )
    full = lambda shp: pl.BlockSpec(shp, lambda b, n: (0,) * len(shp))
    r_sum = pl.pallas_call(
        _enc_body,
        grid=(B, NC // _S),
        in_specs=[
            pl.BlockSpec((1, _S, 1, _PL), lambda b, n: (b, n, 0, 0)),
            pl.BlockSpec((1, 1, 1, _S), lambda b, n: (b, n, 0, 0)),
            full((1, _PL)),
            full((_CC, 9)), full((_CC, 9)), full((_CC, 1)),
            full((9 * _CC, _CC)), full((_CC, 1)),
            full((_CC + 1, _HD)), full((1, _HD)),
            full((_HD, _RD)), full((1, _RD)),
        ],
        out_specs=pl.BlockSpec((1, 1, _RD), lambda b, n: (b, 0, 0)),
        out_shape=jax.ShapeDtypeStruct((B, 1, _RD), f32),
        compiler_params=cparams,
    )(yx, xs, maskext, W1y, W1x, b1c, W2T, b2c,
      We1.astype(bf16), be1r, We2.astype(bf16), be2r)

    cparams2 = pltpu.CompilerParams(
        dimension_semantics=("parallel",),
        vmem_limit_bytes=100 * 1024 * 1024,
    )
    full1 = lambda shp: pl.BlockSpec(shp, lambda b: (0,) * len(shp))
    eps3 = eps.reshape(B, 1, _ZD)
    out_shapes = (
        jax.ShapeDtypeStruct((B, NT, YD), f32),
        jax.ShapeDtypeStruct((B, NT, YD), f32),
        jax.ShapeDtypeStruct((B, 1, _ZD), f32),
        jax.ShapeDtypeStruct((B, 1, _ZD), f32),
    )
    y_mu, y_sigma, mu_c, sigma_c = pl.pallas_call(
        _dec_body,
        grid=(B,),
        in_specs=[
            pl.BlockSpec((1, 1, _RD), lambda b: (b, 0, 0)),
            pl.BlockSpec((1, 1, _ZD), lambda b: (b, 0, 0)),
            pl.BlockSpec((1, NT, 1), lambda b: (b, 0, 0)),
            full1((_RD, _HD)), full1((1, _HD)),
            full1((_HD, _ZD)), full1((1, _ZD)),
            full1((_HD, _ZD)), full1((1, _ZD)),
            full1((1, _HD)), full1((_ZD, _HD)), full1((1, _HD)),
            full1((_HD, _HD)), full1((1, _HD)),
            full1((_HD, YD)), full1((1, YD)),
            full1((_HD, YD)), full1((1, YD)),
        ],
        out_specs=(
            pl.BlockSpec((1, NT, YD), lambda b: (b, 0, 0)),
            pl.BlockSpec((1, NT, YD), lambda b: (b, 0, 0)),
            pl.BlockSpec((1, 1, _ZD), lambda b: (b, 0, 0)),
            pl.BlockSpec((1, 1, _ZD), lambda b: (b, 0, 0)),
        ),
        out_shape=out_shapes,
        compiler_params=cparams2,
    )(r_sum, eps3, x_target,
      Wh.astype(bf16), bh.reshape(1, _HD),
      Wmu.astype(bf16), bmu.reshape(1, _ZD),
      Wsig.astype(bf16), bsig.reshape(1, _ZD),
      Wd1[0:1, :].astype(bf16), Wd1[1:, :].astype(bf16), bd1.reshape(1, _HD),
      Wd2.astype(bf16), bd2.reshape(1, _HD),
      Wdmu.astype(bf16), bdmu.reshape(1, YD),
      Wdsig.astype(bf16), bdsig.reshape(1, YD))

    return y_mu, y_sigma, mu_c.reshape(B, _ZD), sigma_c.reshape(B, _ZD)
